# baseline (speedup 1.0000x reference)
import jax
import jax.numpy as jnp
import numpy as np
from jax.experimental import pallas as pl
from jax.experimental.pallas import tpu as pltpu

_ALPHA = 4


def _linspace_trunc_idx(t: int) -> tuple:
    with jax.ensure_compile_time_eval():
        v = jnp.linspace(0.0, t - 1, t // _ALPHA).astype(jnp.int32)
    return tuple(int(i) for i in np.asarray(v))


def kernel(frames):
    C, T, H, W = frames.shape
    n = T // _ALPHA
    idx = _linspace_trunc_idx(T)

    CH = 48
    nchunk = (C * T) // CH
    DEPTH = 2
    sel = {ch: [] for ch in range(nchunk)}
    for c in range(C):
        for k, s in enumerate(idx):
            g = c * T + s
            sel[g // CH].append((g % CH, c * n + k))

    def body(src, slow, fast, inbuf, slowbuf, in_sem, out_sem, slow_sem):
        def start_in(ch):
            b = ch % DEPTH
            pltpu.make_async_copy(
                src.at[pl.ds(ch * CH, CH)], inbuf.at[b], in_sem.at[b]
            ).start()

        def wait_in(ch):
            b = ch % DEPTH
            pltpu.make_async_copy(
                src.at[pl.ds(ch * CH, CH)], inbuf.at[b], in_sem.at[b]
            ).wait()

        def start_out(ch):
            b = ch % DEPTH
            pltpu.make_async_copy(
                inbuf.at[b], fast.at[pl.ds(ch * CH, CH)], out_sem.at[b]
            ).start()

        def wait_out(ch):
            b = ch % DEPTH
            pltpu.make_async_copy(
                inbuf.at[b], fast.at[pl.ds(ch * CH, CH)], out_sem.at[b]
            ).wait()

        def slow_flush(c):
            return pltpu.make_async_copy(
                slowbuf.at[pl.ds(c * n, n)],
                slow.at[pl.ds(c * n, n)],
                slow_sem.at[c],
            )

        flush_after = {(c * T + T - 1) // CH: c for c in range(C)}

        for ch in range(min(DEPTH - 1, nchunk)):
            start_in(ch)
        for ch in range(nchunk):
            la = ch + DEPTH - 1
            if la < nchunk:
                if la >= DEPTH:
                    wait_out(la - DEPTH)
                start_in(la)
            wait_in(ch)
            start_out(ch)
            for off, j in sel[ch]:
                slowbuf[j] = inbuf[ch % DEPTH, off]
            if ch in flush_after:
                slow_flush(flush_after[ch]).start()
        for ch in range(max(0, nchunk - DEPTH), nchunk):
            wait_out(ch)
        for c in range(C):
            slow_flush(c).wait()

    flat = frames.reshape(C * T, H, W)
    slow, fast = pl.pallas_call(
        body,
        in_specs=[pl.BlockSpec(memory_space=pltpu.MemorySpace.HBM)],
        out_specs=(
            pl.BlockSpec(memory_space=pltpu.MemorySpace.HBM),
            pl.BlockSpec(memory_space=pltpu.MemorySpace.HBM),
        ),
        out_shape=(
            jax.ShapeDtypeStruct((C * n, H, W), frames.dtype),
            jax.ShapeDtypeStruct((C * T, H, W), frames.dtype),
        ),
        scratch_shapes=[
            pltpu.VMEM((DEPTH, CH, H, W), frames.dtype),
            pltpu.VMEM((C * n, H, W), frames.dtype),
            pltpu.SemaphoreType.DMA((DEPTH,)),
            pltpu.SemaphoreType.DMA((DEPTH,)),
            pltpu.SemaphoreType.DMA((C,)),
        ],
    )(flat)
    return (slow.reshape(C, n, H, W), fast.reshape(C, T, H, W))


# speedup vs baseline: 1.0042x; 1.0042x over previous
import jax
import jax.numpy as jnp
import numpy as np
from jax.experimental import pallas as pl
from jax.experimental.pallas import tpu as pltpu

_ALPHA = 4


def _linspace_trunc_idx(t: int) -> tuple:
    with jax.ensure_compile_time_eval():
        v = jnp.linspace(0.0, t - 1, t // _ALPHA).astype(jnp.int32)
    return tuple(int(i) for i in np.asarray(v))


def kernel(frames):
    C, T, H, W = frames.shape
    n = T // _ALPHA
    idx = _linspace_trunc_idx(T)

    CH = 64
    nchunk = (C * T) // CH
    DEPTH = 2
    sel = {ch: [] for ch in range(nchunk)}
    for c in range(C):
        for k, s in enumerate(idx):
            g = c * T + s
            sel[g // CH].append((g % CH, c * n + k))

    def body(src, slow, fast, inbuf, slowbuf, in_sem, out_sem, slow_sem):
        def start_in(ch):
            b = ch % DEPTH
            pltpu.make_async_copy(
                src.at[pl.ds(ch * CH, CH)], inbuf.at[b], in_sem.at[b]
            ).start()

        def wait_in(ch):
            b = ch % DEPTH
            pltpu.make_async_copy(
                src.at[pl.ds(ch * CH, CH)], inbuf.at[b], in_sem.at[b]
            ).wait()

        def start_out(ch):
            b = ch % DEPTH
            pltpu.make_async_copy(
                inbuf.at[b], fast.at[pl.ds(ch * CH, CH)], out_sem.at[b]
            ).start()

        def wait_out(ch):
            b = ch % DEPTH
            pltpu.make_async_copy(
                inbuf.at[b], fast.at[pl.ds(ch * CH, CH)], out_sem.at[b]
            ).wait()

        def slow_flush(c):
            return pltpu.make_async_copy(
                slowbuf.at[pl.ds(c * n, n)],
                slow.at[pl.ds(c * n, n)],
                slow_sem.at[c],
            )

        flush_after = {(c * T + T - 1) // CH: c for c in range(C)}

        for ch in range(min(DEPTH - 1, nchunk)):
            start_in(ch)
        for ch in range(nchunk):
            la = ch + DEPTH - 1
            if la < nchunk:
                if la >= DEPTH:
                    wait_out(la - DEPTH)
                start_in(la)
            wait_in(ch)
            start_out(ch)
            for off, j in sel[ch]:
                slowbuf[j] = inbuf[ch % DEPTH, off]
            if ch in flush_after:
                slow_flush(flush_after[ch]).start()
        for ch in range(max(0, nchunk - DEPTH), nchunk):
            wait_out(ch)
        for c in range(C):
            slow_flush(c).wait()

    flat = frames.reshape(C * T, H, W)
    slow, fast = pl.pallas_call(
        body,
        in_specs=[pl.BlockSpec(memory_space=pltpu.MemorySpace.HBM)],
        out_specs=(
            pl.BlockSpec(memory_space=pltpu.MemorySpace.HBM),
            pl.BlockSpec(memory_space=pltpu.MemorySpace.HBM),
        ),
        out_shape=(
            jax.ShapeDtypeStruct((C * n, H, W), frames.dtype),
            jax.ShapeDtypeStruct((C * T, H, W), frames.dtype),
        ),
        scratch_shapes=[
            pltpu.VMEM((DEPTH, CH, H, W), frames.dtype),
            pltpu.VMEM((C * n, H, W), frames.dtype),
            pltpu.SemaphoreType.DMA((DEPTH,)),
            pltpu.SemaphoreType.DMA((DEPTH,)),
            pltpu.SemaphoreType.DMA((C,)),
        ],
    )(flat)
    return (slow.reshape(C, n, H, W), fast.reshape(C, T, H, W))
